# baseline (speedup 1.0000x reference)
import math

import jax
import jax.numpy as jnp
from jax.experimental import pallas as pl

_TB = 256


def _mog_logprob_kernel(x_ref, mean_ref, scale_ref, wl_ref, out_ref):
    x = x_ref[...]
    wl = wl_ref[...]
    log_w = wl - jax.nn.logsumexp(wl, axis=-1, keepdims=True)

    k_tot = mean_ref.shape[0]
    half_log_2pi = 0.5 * math.log(2.0 * math.pi)

    log2e = 1.0 / math.log(2.0)
    ln2 = math.log(2.0)
    x2 = x * x
    coef = []
    for k in range(k_tot):
        sc = jax.nn.softplus(scale_ref[k, :])[None, :]
        mu = mean_ref[k, :][None, :]
        q = -0.5 / (sc * sc)
        b = -2.0 * q * mu
        a = q * mu * mu - jnp.log(sc) - half_log_2pi + log_w[0:1, k : k + 1]
        coef.append((a * log2e, b * log2e, q * log2e))

    xb = x.astype(jnp.bfloat16)
    xb2 = xb * xb
    m = None
    for a, b, q in coef:
        ab = a.astype(jnp.bfloat16)
        bb = b.astype(jnp.bfloat16)
        qb = q.astype(jnp.bfloat16)
        v = ab + bb * xb + qb * xb2
        m = v if m is None else jnp.maximum(m, v)
    mf = m.astype(jnp.float32)
    s = None
    for a, b, q in coef:
        e = jnp.exp2(a + b * x + q * x2 - mf)
        s = e if s is None else s + e
    out_ref[...] = (mf + jnp.log2(s)) * ln2


def kernel(x, mean_list, scale_list, weight_logits):
    b, z = x.shape
    k = mean_list.shape[-1]
    mean_t = mean_list[0].T
    scale_t = scale_list[0].T

    grid = (b // _TB,)
    return pl.pallas_call(
        _mog_logprob_kernel,
        grid=grid,
        in_specs=[
            pl.BlockSpec((_TB, z), lambda i: (i, 0)),
            pl.BlockSpec((k, z), lambda i: (0, 0)),
            pl.BlockSpec((k, z), lambda i: (0, 0)),
            pl.BlockSpec((1, k), lambda i: (0, 0)),
        ],
        out_specs=pl.BlockSpec((_TB, z), lambda i: (i, 0)),
        out_shape=jax.ShapeDtypeStruct((b, z), x.dtype),
    )(x, mean_t, scale_t, weight_logits)
